# baseline (speedup 1.0000x reference)
import functools

import jax
import jax.numpy as jnp
from jax import lax
from jax.experimental import pallas as pl
from jax.experimental.pallas import tpu as pltpu
from jax.experimental.pallas import tpu_sc as plsc

N = 100000
D = 512
S = 256
NC = 2
NS = 16
L = 16

D_C = D // NC
R_W = 6400
R_LAST = N - (NS - 1) * R_W
CHUNK = 80
GRP = CHUNK // L
V_C = D_C // L
SEG_W = S // NS

IDS_PAD = NS * R_W


def _body(feat_hbm, ids_hbm, out_hbm, part_hbm,
          acc_v, buf_a, buf_b, ids_v, res_v, tmp_a, tmp_b,
          sem_i, sem_a, sem_b):
    c = lax.axis_index("c")
    s = lax.axis_index("s")
    row0 = s * R_W
    col0 = c * D_C
    n_chunks = jnp.where(s == NS - 1, R_LAST // CHUNK, R_W // CHUNK)

    neg_inf = jnp.full((L,), -jnp.inf, dtype=jnp.float32)

    ids_cp = pltpu.make_async_copy(ids_hbm.at[pl.ds(row0, R_W)], ids_v, sem_i)
    ids_cp.start()

    def init_body(i, _):
        for v in range(V_C):
            acc_v[pl.ds(i * D_C + v * L, L)] = neg_inf
        return 0
    lax.fori_loop(0, S, init_body, 0)
    ids_cp.wait()

    def start_chunk(k, buf, sem):
        rs = jnp.minimum(row0 + k * CHUNK, N - CHUNK)
        pltpu.make_async_copy(
            feat_hbm.at[pl.ds(rs, CHUNK), pl.ds(col0, D_C)], buf, sem).start()

    def wait_chunk(buf, sem):
        pltpu.make_async_copy(
            feat_hbm.at[pl.ds(0, CHUNK), pl.ds(0, D_C)], buf, sem).wait()

    def process(buf, ids_base, cur, acc):
        a = list(acc)
        for g in range(GRP):
            seg_vec = ids_v[pl.ds(ids_base + g * L, L)]
            for r in range(L):
                seg = seg_vec[r]
                penalty = jnp.where(seg != cur, jnp.float32(-jnp.inf),
                                    jnp.float32(0.0))
                for v in range(V_C):
                    acc_v[pl.ds(cur * D_C + v * L, L)] = a[v]
                for v in range(V_C):
                    f = buf[g * L + r, pl.ds(v * L, L)]
                    a[v] = jnp.maximum(a[v] + penalty, f)
                cur = seg
        return (cur, tuple(a))

    seg_vec0 = ids_v[pl.ds(0, L)]
    cur0 = seg_vec0[0]
    acc0 = tuple(neg_inf for _ in range(V_C))

    start_chunk(0, buf_a, sem_a)

    def pair_body(k2, carry):
        cur, acc = carry
        ka = 2 * k2
        start_chunk(ka + 1, buf_b, sem_b)
        wait_chunk(buf_a, sem_a)
        cur, acc = process(buf_a, ka * CHUNK, cur, acc)
        start_chunk(ka + 2, buf_a, sem_a)
        wait_chunk(buf_b, sem_b)
        cur, acc = process(buf_b, (ka + 1) * CHUNK, cur, acc)
        return (cur, acc)

    cur, acc = lax.fori_loop(0, n_chunks // 2, pair_body, (cur0, acc0))
    wait_chunk(buf_a, sem_a)
    for v in range(V_C):
        acc_v[pl.ds(cur * D_C + v * L, L)] = acc[v]

    pltpu.sync_copy(acc_v, part_hbm.at[c, s])
    plsc.subcore_barrier()

    seg0 = s * SEG_W

    def start_slab(p, buf, sem):
        pltpu.make_async_copy(
            part_hbm.at[c, p, pl.ds(seg0 * D_C, SEG_W * D_C)], buf,
            sem).start()

    def wait_slab(buf, sem):
        pltpu.make_async_copy(
            part_hbm.at[c, 0, pl.ds(0, SEG_W * D_C)], buf, sem).wait()

    start_slab(0, tmp_a, sem_a)
    start_slab(1, tmp_b, sem_b)
    wait_slab(tmp_a, sem_a)

    def cp_body(i, _):
        for v in range(V_C):
            res_v[i, pl.ds(v * L, L)] = tmp_a[pl.ds(i * D_C + v * L, L)]
        return 0
    lax.fori_loop(0, SEG_W, cp_body, 0)

    for p in range(1, NS):
        buf, sem = (tmp_b, sem_b) if p % 2 == 1 else (tmp_a, sem_a)
        if p + 1 < NS:
            nbuf, nsem = (tmp_b, sem_b) if (p + 1) % 2 == 1 else (tmp_a, sem_a)
            start_slab(p + 1, nbuf, nsem)
        wait_slab(buf, sem)

        def red_body(i, _, buf=buf):
            for v in range(V_C):
                a = res_v[i, pl.ds(v * L, L)]
                b = buf[pl.ds(i * D_C + v * L, L)]
                res_v[i, pl.ds(v * L, L)] = jnp.maximum(a, b)
            return 0
        lax.fori_loop(0, SEG_W, red_body, 0)

    pltpu.sync_copy(res_v, out_hbm.at[pl.ds(seg0, SEG_W), pl.ds(col0, D_C)])


@jax.jit
def _segment_max_sc(features, ids_padded):
    mesh = plsc.VectorSubcoreMesh(core_axis_name="c", subcore_axis_name="s")
    f = functools.partial(
        pl.kernel,
        out_type=(
            jax.ShapeDtypeStruct((S, D), jnp.float32),
            jax.ShapeDtypeStruct((NC, NS, S * D_C), jnp.float32),
        ),
        mesh=mesh,
        scratch_types=[
            pltpu.VMEM((S * D_C,), jnp.float32),
            pltpu.VMEM((CHUNK, D_C), jnp.float32),
            pltpu.VMEM((CHUNK, D_C), jnp.float32),
            pltpu.VMEM((R_W,), jnp.int32),
            pltpu.VMEM((SEG_W, D_C), jnp.float32),
            pltpu.VMEM((SEG_W * D_C,), jnp.float32),
            pltpu.VMEM((SEG_W * D_C,), jnp.float32),
            pltpu.SemaphoreType.DMA,
            pltpu.SemaphoreType.DMA,
            pltpu.SemaphoreType.DMA,
        ],
    )(_body)
    out, _ = f(features, ids_padded)
    return out


def kernel(features, segment_ids, num_segments):
    ids = segment_ids.astype(jnp.int32)
    ids_padded = jnp.pad(ids, (0, IDS_PAD - N), constant_values=S - 1)
    return _segment_max_sc(features, ids_padded)


# speedup vs baseline: 2.3890x; 2.3890x over previous
import functools

import jax
import jax.numpy as jnp
from jax import lax
from jax.experimental import pallas as pl
from jax.experimental.pallas import tpu as pltpu
from jax.experimental.pallas import tpu_sc as plsc

N = 100000
D = 512
S = 256
NC = 2
NS = 16
L = 16

D_C = D // NC
R_W = 6400
R_LAST = N - (NS - 1) * R_W
CHUNK = 80
GRP = CHUNK // L
V_C = D_C // L
SEG_W = S // NS

IDS_PAD = NS * R_W


def _body(feat_hbm, ids_hbm, out_hbm, part_hbm,
          acc_v, buf_a, buf_b, ids_v, res_v, tmp_a, tmp_b,
          sem_i, sem_a, sem_b):
    c = lax.axis_index("c")
    s = lax.axis_index("s")
    row0 = s * R_W
    col0 = c * D_C
    n_chunks = jnp.where(s == NS - 1, R_LAST // CHUNK, R_W // CHUNK)

    neg_inf = jnp.full((L,), -jnp.inf, dtype=jnp.float32)

    @pl.when(s < NS - 1)
    def _():
        pltpu.make_async_copy(ids_hbm.at[pl.ds(row0, R_W)], ids_v,
                              sem_i).start()

    @pl.when(s == NS - 1)
    def _():
        pltpu.make_async_copy(ids_hbm.at[pl.ds(row0, R_LAST)],
                              ids_v.at[pl.ds(0, R_LAST)], sem_i).start()

    def init_body(i, _):
        for v in range(V_C):
            acc_v[pl.ds(i * D_C + v * L, L)] = neg_inf
        return 0
    lax.fori_loop(0, S, init_body, 0)

    @pl.when(s < NS - 1)
    def _():
        pltpu.make_async_copy(ids_hbm.at[pl.ds(row0, R_W)], ids_v,
                              sem_i).wait()

    @pl.when(s == NS - 1)
    def _():
        pltpu.make_async_copy(ids_hbm.at[pl.ds(row0, R_LAST)],
                              ids_v.at[pl.ds(0, R_LAST)], sem_i).wait()

    def start_chunk(k, buf, sem):
        rs = jnp.minimum(row0 + k * CHUNK, N - CHUNK)
        pltpu.make_async_copy(
            feat_hbm.at[pl.ds(rs, CHUNK), pl.ds(col0, D_C)], buf, sem).start()

    def wait_chunk(buf, sem):
        pltpu.make_async_copy(
            feat_hbm.at[pl.ds(0, CHUNK), pl.ds(0, D_C)], buf, sem).wait()

    def process(buf, ids_base, cur, acc):
        def grp_body(g, carry):
            cur, acc = carry
            seg_vec = ids_v[pl.ds(ids_base + g * L, L)]
            a = list(acc)
            for r in range(L):
                seg = seg_vec[r]
                penalty = jnp.where(seg != cur, jnp.float32(-jnp.inf),
                                    jnp.float32(0.0))
                for v in range(V_C):
                    acc_v[pl.ds(cur * D_C + v * L, L)] = a[v]
                for v in range(V_C):
                    f = buf[g * L + r, pl.ds(v * L, L)]
                    a[v] = jnp.maximum(a[v] + penalty, f)
                cur = seg
            return (cur, tuple(a))

        return lax.fori_loop(0, GRP, grp_body, (cur, acc))

    seg_vec0 = ids_v[pl.ds(0, L)]
    cur0 = seg_vec0[0]
    acc0 = tuple(neg_inf for _ in range(V_C))

    start_chunk(0, buf_a, sem_a)

    def pair_body(k2, carry):
        cur, acc = carry
        ka = 2 * k2
        start_chunk(ka + 1, buf_b, sem_b)
        wait_chunk(buf_a, sem_a)
        cur, acc = process(buf_a, ka * CHUNK, cur, acc)
        start_chunk(ka + 2, buf_a, sem_a)
        wait_chunk(buf_b, sem_b)
        cur, acc = process(buf_b, (ka + 1) * CHUNK, cur, acc)
        return (cur, acc)

    cur, acc = lax.fori_loop(0, n_chunks // 2, pair_body, (cur0, acc0))
    wait_chunk(buf_a, sem_a)
    for v in range(V_C):
        acc_v[pl.ds(cur * D_C + v * L, L)] = acc[v]

    pltpu.sync_copy(acc_v, part_hbm.at[c, s])
    plsc.subcore_barrier()

    seg0 = s * SEG_W

    def start_slab(p, buf, sem):
        pltpu.make_async_copy(
            part_hbm.at[c, p, pl.ds(seg0 * D_C, SEG_W * D_C)], buf,
            sem).start()

    def wait_slab(buf, sem):
        pltpu.make_async_copy(
            part_hbm.at[c, 0, pl.ds(0, SEG_W * D_C)], buf, sem).wait()

    start_slab(0, tmp_a, sem_a)
    start_slab(1, tmp_b, sem_b)
    wait_slab(tmp_a, sem_a)

    def cp_body(i, _):
        for v in range(V_C):
            res_v[i, pl.ds(v * L, L)] = tmp_a[pl.ds(i * D_C + v * L, L)]
        return 0
    lax.fori_loop(0, SEG_W, cp_body, 0)

    for p in range(1, NS):
        buf, sem = (tmp_b, sem_b) if p % 2 == 1 else (tmp_a, sem_a)
        if p + 1 < NS:
            nbuf, nsem = (tmp_b, sem_b) if (p + 1) % 2 == 1 else (tmp_a, sem_a)
            start_slab(p + 1, nbuf, nsem)
        wait_slab(buf, sem)

        def red_body(i, _, buf=buf):
            for v in range(V_C):
                a = res_v[i, pl.ds(v * L, L)]
                b = buf[pl.ds(i * D_C + v * L, L)]
                res_v[i, pl.ds(v * L, L)] = jnp.maximum(a, b)
            return 0
        lax.fori_loop(0, SEG_W, red_body, 0)

    pltpu.sync_copy(res_v, out_hbm.at[pl.ds(seg0, SEG_W), pl.ds(col0, D_C)])


@jax.jit
def _segment_max_sc(features, ids_padded):
    mesh = plsc.VectorSubcoreMesh(core_axis_name="c", subcore_axis_name="s")
    f = functools.partial(
        pl.kernel,
        out_type=(
            jax.ShapeDtypeStruct((S, D), jnp.float32),
            jax.ShapeDtypeStruct((NC, NS, S * D_C), jnp.float32),
        ),
        mesh=mesh,
        scratch_types=[
            pltpu.VMEM((S * D_C,), jnp.float32),
            pltpu.VMEM((CHUNK, D_C), jnp.float32),
            pltpu.VMEM((CHUNK, D_C), jnp.float32),
            pltpu.VMEM((R_W,), jnp.int32),
            pltpu.VMEM((SEG_W, D_C), jnp.float32),
            pltpu.VMEM((SEG_W * D_C,), jnp.float32),
            pltpu.VMEM((SEG_W * D_C,), jnp.float32),
            pltpu.SemaphoreType.DMA,
            pltpu.SemaphoreType.DMA,
            pltpu.SemaphoreType.DMA,
        ],
    )(_body)
    out, _ = f(features, ids_padded)
    return out


def kernel(features, segment_ids, num_segments):
    return _segment_max_sc(features, segment_ids.astype(jnp.int32))
